# baseline (speedup 1.0000x reference)
import jax
import jax.numpy as jnp
from jax import lax
from jax.experimental import pallas as pl
from jax.experimental.pallas import tpu as pltpu
from jax.experimental.pallas import tpu_sc as plsc

B = 16384
D = 64
NC = 2
NS = 16
NW = NC * NS
BPW = B // NW
WAVE = 2

_MESH = plsc.VectorSubcoreMesh(core_axis_name="c", subcore_axis_name="s",
                               num_cores=NC, num_subcores=NS)


def _worker_base():
    return (lax.axis_index("s") * NC + lax.axis_index("c")) * BPW


def _dot_body(uid_hbm, pid_hbm, nid_hbm, utabT_hbm, itabT_hbm, diff_hbm,
              uidx, pidx, nidx, ubuf, pbuf, nbuf, diffv, sem):
    base = _worker_base()
    pltpu.sync_copy(uid_hbm.at[pl.ds(base, BPW)], uidx)
    pltpu.sync_copy(pid_hbm.at[pl.ds(base, BPW)], pidx)
    pltpu.sync_copy(nid_hbm.at[pl.ds(base, BPW)], nidx)
    iota16 = lax.iota(jnp.int32, 16)

    nwave = 16 // WAVE

    def group_body(g, carry):
        uv = uidx[pl.ds(g * 16, 16)]
        pv = pidx[pl.ds(g * 16, 16)]
        nv = nidx[pl.ds(g * 16, 16)]

        def issue(w):
            par = w % 2
            copies, lanes = [], []
            for j in range(WAVE):
                k = w * WAVE + j
                ub, pb_, nb_ = uv[k], pv[k], nv[k]
                us = pl.multiple_of((ub >> 7) << 7, 128)
                ps = pl.multiple_of((pb_ >> 7) << 7, 128)
                ns = pl.multiple_of((nb_ >> 7) << 7, 128)
                s = sem.at[par]
                copies.append(pltpu.async_copy(
                    utabT_hbm.at[:, pl.ds(us, 128)], ubuf.at[par, j], s))
                copies.append(pltpu.async_copy(
                    itabT_hbm.at[:, pl.ds(ps, 128)], pbuf.at[par, j], s))
                copies.append(pltpu.async_copy(
                    itabT_hbm.at[:, pl.ds(ns, 128)], nbuf.at[par, j], s))
                lanes.append((ub & 127, pb_ & 127, nb_ & 127))
            return copies, lanes

        def extract(w, copies, lanes, diff16):
            par = w % 2
            for c in copies:
                c.wait()
            for j in range(WAVE):
                k = w * WAVE + j
                ul, pl_, nl = lanes[j]
                pj16 = jnp.full((16,), par, jnp.int32)
                j16 = jnp.full((16,), j, jnp.int32)
                ul16 = jnp.full((16,), ul, jnp.int32)
                pl16 = jnp.full((16,), pl_, jnp.int32)
                nl16 = jnp.full((16,), nl, jnp.int32)
                acc = jnp.zeros((16,), jnp.float32)
                for dg in range(D // 16):
                    d16 = dg * 16 + iota16
                    uu = plsc.load_gather(ubuf, [pj16, j16, d16, ul16])
                    pp = plsc.load_gather(pbuf, [pj16, j16, d16, pl16])
                    nn = plsc.load_gather(nbuf, [pj16, j16, d16, nl16])
                    acc = acc + uu * (pp - nn)
                s = jnp.sum(acc)
                diff16 = jnp.where(iota16 == k, s, diff16)
            return diff16

        diff16 = jnp.zeros((16,), jnp.float32)
        pend = issue(0)
        for w in range(nwave):
            nxt = issue(w + 1) if w + 1 < nwave else None
            diff16 = extract(w, pend[0], pend[1], diff16)
            pend = nxt
        diffv[pl.ds(g * 16, 16)] = diff16
        return carry

    lax.fori_loop(0, BPW // 16, group_body, 0)
    pltpu.sync_copy(diffv, diff_hbm.at[pl.ds(base, BPW)])


_dot_call = pl.kernel(
    _dot_body,
    out_type=jax.ShapeDtypeStruct((B,), jnp.float32),
    mesh=_MESH,
    scratch_types=[
        pltpu.VMEM((BPW,), jnp.int32),
        pltpu.VMEM((BPW,), jnp.int32),
        pltpu.VMEM((BPW,), jnp.int32),
        pltpu.VMEM((2, WAVE, D, 128), jnp.float32),
        pltpu.VMEM((2, WAVE, D, 128), jnp.float32),
        pltpu.VMEM((2, WAVE, D, 128), jnp.float32),
        pltpu.VMEM((BPW,), jnp.float32),
        pltpu.SemaphoreType.DMA((2,)),
    ],
    compiler_params=pltpu.CompilerParams(needs_layout_passes=False),
)


def _bias_body(pid_hbm, nid_hbm, ibias_hbm, bdiff_hbm,
               pidx, nidx, pb, nb, bdiffv, sem):
    base = _worker_base()
    pltpu.sync_copy(pid_hbm.at[pl.ds(base, BPW)], pidx)
    pltpu.sync_copy(nid_hbm.at[pl.ds(base, BPW)], nidx)
    copies = []
    for j in range(BPW // 128):
        sl = pl.ds(j * 128, 128)
        copies.append(pltpu.async_copy(ibias_hbm.at[pidx.at[sl]], pb.at[sl], sem))
        copies.append(pltpu.async_copy(ibias_hbm.at[nidx.at[sl]], nb.at[sl], sem))
    for c in copies:
        c.wait()

    def group_body(i, carry):
        sl = pl.ds(i * 16, 16)
        bdiffv[sl] = pb[sl] - nb[sl]
        return carry

    lax.fori_loop(0, BPW // 16, group_body, 0)
    pltpu.sync_copy(bdiffv, bdiff_hbm.at[pl.ds(base, BPW)])


_bias_call = pl.kernel(
    _bias_body,
    out_type=jax.ShapeDtypeStruct((B,), jnp.float32),
    mesh=_MESH,
    scratch_types=[
        pltpu.VMEM((BPW,), jnp.int32),
        pltpu.VMEM((BPW,), jnp.int32),
        pltpu.VMEM((BPW,), jnp.float32),
        pltpu.VMEM((BPW,), jnp.float32),
        pltpu.VMEM((BPW,), jnp.float32),
        pltpu.SemaphoreType.DMA,
    ],
    compiler_params=pltpu.CompilerParams(needs_layout_passes=False,
                                         use_tc_tiling_on_sc=False),
)


def _loss_body(diff_ref, bd_ref, out_ref):
    x = diff_ref[...] + bd_ref[...]
    sp = jnp.maximum(-x, 0.0) + jnp.log1p(jnp.exp(-jnp.abs(x)))
    out_ref[...] = jnp.sum(sp).reshape(1, 1)


_loss_call = pl.pallas_call(
    _loss_body,
    out_shape=jax.ShapeDtypeStruct((1, 1), jnp.float32),
)


def kernel(user_id, p_item_id, n_item_id, user_table, item_table, item_bias):
    uid = user_id.astype(jnp.int32)
    pid = p_item_id.astype(jnp.int32)
    nid = n_item_id.astype(jnp.int32)
    diff = _dot_call(uid, pid, nid, user_table.T, item_table.T)
    bdiff = _bias_call(pid, nid, item_bias.reshape(-1))
    loss = _loss_call(diff.reshape(B // 128, 128),
                      bdiff.reshape(B // 128, 128))
    return loss[0, 0]


# speedup vs baseline: 1.0510x; 1.0510x over previous
import jax
import jax.numpy as jnp
from jax import lax
from jax.experimental import pallas as pl
from jax.experimental.pallas import tpu as pltpu
from jax.experimental.pallas import tpu_sc as plsc

B = 16384
D = 64
NC = 2
NS = 16
NW = NC * NS
BPW = B // NW
WAVE = 4

_MESH = plsc.VectorSubcoreMesh(core_axis_name="c", subcore_axis_name="s",
                               num_cores=NC, num_subcores=NS)


def _worker_base():
    return (lax.axis_index("s") * NC + lax.axis_index("c")) * BPW


def _dot_body(uid_hbm, pid_hbm, nid_hbm, utabT_hbm, itabT_hbm, diff_hbm,
              uidx, pidx, nidx, ubuf, pbuf, nbuf, diffv, sem):
    base = _worker_base()
    pltpu.sync_copy(uid_hbm.at[pl.ds(base, BPW)], uidx)
    pltpu.sync_copy(pid_hbm.at[pl.ds(base, BPW)], pidx)
    pltpu.sync_copy(nid_hbm.at[pl.ds(base, BPW)], nidx)
    iota16 = lax.iota(jnp.int32, 16)

    def group_body(g, carry):
        uv = uidx[pl.ds(g * 16, 16)]
        pv = pidx[pl.ds(g * 16, 16)]
        nv = nidx[pl.ds(g * 16, 16)]
        diff16 = jnp.zeros((16,), jnp.float32)
        for w in range(16 // WAVE):
            copies = []
            lanes = []
            for j in range(WAVE):
                k = w * WAVE + j
                ub, pb_, nb_ = uv[k], pv[k], nv[k]
                us = pl.multiple_of((ub >> 7) << 7, 128)
                ps = pl.multiple_of((pb_ >> 7) << 7, 128)
                ns = pl.multiple_of((nb_ >> 7) << 7, 128)
                copies.append(pltpu.async_copy(
                    utabT_hbm.at[:, pl.ds(us, 128)], ubuf.at[j], sem))
                copies.append(pltpu.async_copy(
                    itabT_hbm.at[:, pl.ds(ps, 128)], pbuf.at[j], sem))
                copies.append(pltpu.async_copy(
                    itabT_hbm.at[:, pl.ds(ns, 128)], nbuf.at[j], sem))
                lanes.append((ub & 127, pb_ & 127, nb_ & 127))
            for c in copies:
                c.wait()
            for j in range(WAVE):
                k = w * WAVE + j
                ul, pl_, nl = lanes[j]
                j16 = jnp.full((16,), j, jnp.int32)
                ul16 = jnp.full((16,), ul, jnp.int32)
                pl16 = jnp.full((16,), pl_, jnp.int32)
                nl16 = jnp.full((16,), nl, jnp.int32)
                acc = jnp.zeros((16,), jnp.float32)
                for dg in range(D // 16):
                    d16 = dg * 16 + iota16
                    uu = plsc.load_gather(ubuf, [j16, d16, ul16])
                    pp = plsc.load_gather(pbuf, [j16, d16, pl16])
                    nn = plsc.load_gather(nbuf, [j16, d16, nl16])
                    acc = acc + uu * (pp - nn)
                s = jnp.sum(acc)
                diff16 = jnp.where(iota16 == k, s, diff16)
        diffv[pl.ds(g * 16, 16)] = diff16
        return carry

    lax.fori_loop(0, BPW // 16, group_body, 0)
    pltpu.sync_copy(diffv, diff_hbm.at[pl.ds(base, BPW)])


_dot_call = pl.kernel(
    _dot_body,
    out_type=jax.ShapeDtypeStruct((B,), jnp.float32),
    mesh=_MESH,
    scratch_types=[
        pltpu.VMEM((BPW,), jnp.int32),
        pltpu.VMEM((BPW,), jnp.int32),
        pltpu.VMEM((BPW,), jnp.int32),
        pltpu.VMEM((WAVE, D, 128), jnp.float32),
        pltpu.VMEM((WAVE, D, 128), jnp.float32),
        pltpu.VMEM((WAVE, D, 128), jnp.float32),
        pltpu.VMEM((BPW,), jnp.float32),
        pltpu.SemaphoreType.DMA,
    ],
    compiler_params=pltpu.CompilerParams(needs_layout_passes=False),
)


def _bias_body(pid_hbm, nid_hbm, ibias_hbm, bdiff_hbm,
               pidx, nidx, pb, nb, bdiffv, sem):
    base = _worker_base()
    pltpu.sync_copy(pid_hbm.at[pl.ds(base, BPW)], pidx)
    pltpu.sync_copy(nid_hbm.at[pl.ds(base, BPW)], nidx)
    copies = []
    for j in range(BPW // 128):
        sl = pl.ds(j * 128, 128)
        copies.append(pltpu.async_copy(ibias_hbm.at[pidx.at[sl]], pb.at[sl], sem))
        copies.append(pltpu.async_copy(ibias_hbm.at[nidx.at[sl]], nb.at[sl], sem))
    for c in copies:
        c.wait()

    def group_body(i, carry):
        sl = pl.ds(i * 16, 16)
        bdiffv[sl] = pb[sl] - nb[sl]
        return carry

    lax.fori_loop(0, BPW // 16, group_body, 0)
    pltpu.sync_copy(bdiffv, bdiff_hbm.at[pl.ds(base, BPW)])


_bias_call = pl.kernel(
    _bias_body,
    out_type=jax.ShapeDtypeStruct((B,), jnp.float32),
    mesh=_MESH,
    scratch_types=[
        pltpu.VMEM((BPW,), jnp.int32),
        pltpu.VMEM((BPW,), jnp.int32),
        pltpu.VMEM((BPW,), jnp.float32),
        pltpu.VMEM((BPW,), jnp.float32),
        pltpu.VMEM((BPW,), jnp.float32),
        pltpu.SemaphoreType.DMA,
    ],
    compiler_params=pltpu.CompilerParams(needs_layout_passes=False,
                                         use_tc_tiling_on_sc=False),
)


def _loss_body(diff_ref, bd_ref, out_ref):
    x = diff_ref[...] + bd_ref[...]
    sp = jnp.maximum(-x, 0.0) + jnp.log1p(jnp.exp(-jnp.abs(x)))
    out_ref[...] = jnp.sum(sp).reshape(1, 1)


_loss_call = pl.pallas_call(
    _loss_body,
    out_shape=jax.ShapeDtypeStruct((1, 1), jnp.float32),
)


def kernel(user_id, p_item_id, n_item_id, user_table, item_table, item_bias):
    uid = user_id.astype(jnp.int32)
    pid = p_item_id.astype(jnp.int32)
    nid = n_item_id.astype(jnp.int32)
    diff = _dot_call(uid, pid, nid, user_table.T, item_table.T)
    bdiff = _bias_call(pid, nid, item_bias.reshape(-1))
    loss = _loss_call(diff.reshape(B // 128, 128),
                      bdiff.reshape(B // 128, 128))
    return loss[0, 0]
